# Initial kernel scaffold; baseline (speedup 1.0000x reference)
import functools

import jax
import jax.numpy as jnp
from jax import lax
from jax.experimental import pallas as pl
from jax.experimental.pallas import tpu as pltpu
from jax.experimental.pallas import tpu_sc as plsc

N = 12
E = 768
K = 125
S = 8
KS = 5
NW = 32
EPW = E // NW


def _tc_a_body(x_ref, w1t_ref, eat_ref, src_ref, xw_ref, basis_ref, fidx_ref):
    xw_ref[...] = jnp.dot(x_ref[...], w1t_ref[...],
                          preferred_element_type=jnp.float32)
    v = eat_ref[...] * (KS - 1.0)
    bot = jnp.clip(jnp.floor(v), 0.0, KS - 2.0)
    frac = v - bot
    boti = bot.astype(jnp.int32)
    src = src_ref[...]
    for s in range(S):
        b = jnp.ones((1, E), jnp.float32)
        wi = jnp.zeros((1, E), jnp.int32)
        stride = 1
        for d in range(3):
            bit = (s >> d) & 1
            fd = frac[d:d + 1, :]
            b = b * (fd if bit else 1.0 - fd)
            wi = wi + (boti[d:d + 1, :] + bit) * stride
            stride *= KS
        basis_ref[s:s + 1, :] = b
        fidx_ref[s:s + 1, :] = src * K + wi


def _tc_a(x, w1t, eat, src):
    return pl.pallas_call(
        _tc_a_body,
        out_shape=(
            jax.ShapeDtypeStruct((N, K * 32), jnp.float32),
            jax.ShapeDtypeStruct((S, E), jnp.float32),
            jax.ShapeDtypeStruct((S, E), jnp.int32),
        ),
    )(x, w1t, eat, src)


def _sc_b_body(fout, xw_hbm, fidx_hbm, basis_hbm, out_hbm,
               fidx_v, basis_v, rows_v, msg_v, sem):
    wid = lax.axis_index("s") * 2 + lax.axis_index("c")
    e0 = wid * EPW
    pltpu.sync_copy(fidx_hbm.at[:, pl.ds(e0, EPW)], fidx_v)
    pltpu.sync_copy(basis_hbm.at[:, pl.ds(e0, EPW)], basis_v)
    copies = [
        pltpu.async_copy(xw_hbm.at[fidx_v.at[s]], rows_v.at[s], sem)
        for s in range(S)
    ]
    for c in copies:
        c.wait()
    for e in range(EPW):
        spl = [
            plsc.load_gather(
                basis_v,
                [jnp.full((16,), s, jnp.int32), jnp.full((16,), e, jnp.int32)],
            )
            for s in range(S)
        ]
        for c in range(fout // 16):
            acc = jnp.zeros((16,), jnp.float32)
            for s in range(S):
                acc = acc + spl[s] * rows_v[s, e, pl.ds(c * 16, 16)]
            msg_v[e, pl.ds(c * 16, 16)] = acc
    pltpu.sync_copy(msg_v, out_hbm.at[pl.ds(e0, EPW), :])


def _sc_b(fout, xwflat, fidx, basis):
    mesh = plsc.VectorSubcoreMesh(core_axis_name="c", subcore_axis_name="s")
    kfn = functools.partial(
        pl.kernel,
        mesh=mesh,
        out_type=jax.ShapeDtypeStruct((E, fout), jnp.float32),
        scratch_types=[
            pltpu.VMEM((S, EPW), jnp.int32),
            pltpu.VMEM((S, EPW), jnp.float32),
            pltpu.VMEM((S, EPW, fout), jnp.float32),
            pltpu.VMEM((EPW, fout), jnp.float32),
            pltpu.SemaphoreType.DMA,
        ],
    )(functools.partial(_sc_b_body, fout))
    return kfn(xwflat, fidx, basis)


def _elu(a):
    return jnp.where(a > 0.0, a, jnp.exp(jnp.minimum(a, 0.0)) - 1.0)


def _segmean(msg, dst):
    onehot = jnp.where(
        lax.broadcasted_iota(jnp.int32, (N, E), 0) == dst, 1.0, 0.0)
    cnt = jnp.sum(onehot, axis=1, keepdims=True)
    agg = jnp.dot(onehot, msg, preferred_element_type=jnp.float32)
    return agg / jnp.maximum(cnt, 1.0)


def _tc_c_body(msg_ref, dst_ref, x_ref, root1_ref, b1_ref, w2t_ref,
               xw2_ref, h1_ref):
    agg = _segmean(msg_ref[...], dst_ref[...])
    h1 = _elu(agg + jnp.dot(x_ref[...], root1_ref[...],
                            preferred_element_type=jnp.float32)
              + b1_ref[...])
    h1_ref[...] = h1
    xw2_ref[...] = jnp.dot(h1, w2t_ref[...],
                           preferred_element_type=jnp.float32)


def _tc_c(msg1, dst, x, root1, b1, w2t):
    return pl.pallas_call(
        _tc_c_body,
        out_shape=(
            jax.ShapeDtypeStruct((N, K * 64), jnp.float32),
            jax.ShapeDtypeStruct((N, 32), jnp.float32),
        ),
    )(msg1, dst, x, root1, b1, w2t)


def _tc_e_body(msg_ref, dst_ref, h1_ref, root2_ref, b2_ref,
               fc1w_ref, fc1b_ref, fc2w_ref, fc2b_ref, out_ref):
    agg = _segmean(msg_ref[...], dst_ref[...])
    h2 = _elu(agg + jnp.dot(h1_ref[...], root2_ref[...],
                            preferred_element_type=jnp.float32)
              + b2_ref[...])
    y = fc1b_ref[...].reshape(1, -1)
    for n in range(N):
        y = y + jnp.dot(h2[n:n + 1, :], fc1w_ref[pl.ds(n * 64, 64), :],
                        preferred_element_type=jnp.float32)
    z = jnp.dot(y, fc2w_ref[...],
                preferred_element_type=jnp.float32) + fc2b_ref[...]
    m = jnp.max(z, axis=1, keepdims=True)
    out_ref[...] = z - (m + jnp.log(jnp.sum(jnp.exp(z - m), axis=1,
                                            keepdims=True)))


def _tc_e(msg2, dst, h1, root2, b2, fc1_w, fc1_b, fc2_w, fc2_b):
    return pl.pallas_call(
        _tc_e_body,
        out_shape=jax.ShapeDtypeStruct((1, 2), jnp.float32),
    )(msg2, dst, h1, root2, b2, fc1_w, fc1_b, fc2_w, fc2_b)


def kernel(x, edge_index, edge_attr, W1, root1, b1, W2, root2, b2,
           fc1_w, fc1_b, fc2_w, fc2_b):
    src = edge_index[0].reshape(1, E)
    dst = edge_index[1].reshape(1, E)
    eat = edge_attr.T
    w1t = W1.transpose(1, 0, 2).reshape(128, K * 32)
    w2t = W2.transpose(1, 0, 2).reshape(32, K * 64)

    xw1, basis, fidx = _tc_a(x, w1t, eat, src)
    msg1 = _sc_b(32, xw1.reshape(N * K, 32), fidx, basis)
    xw2, h1 = _tc_c(msg1, dst, x, root1, b1.reshape(1, 32), w2t)
    msg2 = _sc_b(64, xw2.reshape(N * K, 64), fidx, basis)
    return _tc_e(msg2, dst, h1, root2, b2.reshape(1, 64),
                 fc1_w, fc1_b, fc2_w, fc2_b)


# speedup vs baseline: 4.2524x; 4.2524x over previous
import functools

import jax
import jax.numpy as jnp
from jax import lax
from jax.experimental import pallas as pl
from jax.experimental.pallas import tpu as pltpu
from jax.experimental.pallas import tpu_sc as plsc

N = 12
E = 768
K = 125
S = 8
KS = 5
NW = 32
EPW = E // NW


def _tc_a_body(x_ref, w1t_ref, eat_ref, src_ref, xw_ref, basis_ref, fidx_ref):
    xw_ref[...] = jnp.dot(x_ref[...], w1t_ref[...],
                          preferred_element_type=jnp.float32)
    v = eat_ref[...] * (KS - 1.0)
    bot = jnp.clip(jnp.floor(v), 0.0, KS - 2.0)
    frac = v - bot
    boti = bot.astype(jnp.int32)
    src = src_ref[...]
    for s in range(S):
        b = jnp.ones((1, E), jnp.float32)
        wi = jnp.zeros((1, E), jnp.int32)
        stride = 1
        for d in range(3):
            bit = (s >> d) & 1
            fd = frac[d:d + 1, :]
            b = b * (fd if bit else 1.0 - fd)
            wi = wi + (boti[d:d + 1, :] + bit) * stride
            stride *= KS
        basis_ref[s:s + 1, :] = b
        fidx_ref[s:s + 1, :] = src * K + wi


def _tc_a(x, w1t, eat, src):
    return pl.pallas_call(
        _tc_a_body,
        out_shape=(
            jax.ShapeDtypeStruct((N, K * 32), jnp.float32),
            jax.ShapeDtypeStruct((S, E), jnp.float32),
            jax.ShapeDtypeStruct((S, E), jnp.int32),
        ),
    )(x, w1t, eat, src)


def _sc_b_body(fout, xw_hbm, fidx_hbm, basis_hbm, out_hbm,
               fidx_v, basis_v, rows_v, msg_v, sem):
    wid = lax.axis_index("s") * 2 + lax.axis_index("c")
    e0 = wid * EPW
    pltpu.sync_copy(fidx_hbm.at[:, pl.ds(e0, EPW)], fidx_v)
    pltpu.sync_copy(basis_hbm.at[:, pl.ds(e0, EPW)], basis_v)
    copies = [
        pltpu.async_copy(xw_hbm.at[fidx_v.at[s]], rows_v.at[s], sem)
        for s in range(S)
    ]
    for c in copies:
        c.wait()
    for e in range(EPW):
        spl = [
            plsc.load_gather(
                basis_v,
                [jnp.full((16,), s, jnp.int32), jnp.full((16,), e, jnp.int32)],
            )
            for s in range(S)
        ]
        for c in range(fout // 16):
            acc = jnp.zeros((16,), jnp.float32)
            for s in range(S):
                acc = acc + spl[s] * rows_v[s, e, pl.ds(c * 16, 16)]
            msg_v[e, pl.ds(c * 16, 16)] = acc
    pltpu.sync_copy(msg_v, out_hbm.at[pl.ds(e0, EPW), :])


def _sc_b(fout, xwflat, fidx, basis):
    mesh = plsc.VectorSubcoreMesh(core_axis_name="c", subcore_axis_name="s")
    kfn = functools.partial(
        pl.kernel,
        mesh=mesh,
        out_type=jax.ShapeDtypeStruct((E, fout), jnp.float32),
        scratch_types=[
            pltpu.VMEM((S, EPW), jnp.int32),
            pltpu.VMEM((S, EPW), jnp.float32),
            pltpu.VMEM((S, EPW, fout), jnp.float32),
            pltpu.VMEM((EPW, fout), jnp.float32),
            pltpu.SemaphoreType.DMA,
        ],
        compiler_params=pltpu.CompilerParams(use_tc_tiling_on_sc=False,
                                             needs_layout_passes=False),
    )(functools.partial(_sc_b_body, fout))
    return kfn(xwflat, fidx, basis)


def _elu(a):
    return jnp.where(a > 0.0, a, jnp.exp(jnp.minimum(a, 0.0)) - 1.0)


def _segmean(msg, dst):
    onehot = jnp.where(
        lax.broadcasted_iota(jnp.int32, (N, E), 0) == dst, 1.0, 0.0)
    cnt = jnp.sum(onehot, axis=1, keepdims=True)
    agg = jnp.dot(onehot, msg, preferred_element_type=jnp.float32)
    return agg / jnp.maximum(cnt, 1.0)


def _tc_c_body(msg_ref, dst_ref, x_ref, root1_ref, b1_ref, w2t_ref,
               xw2_ref, h1_ref):
    agg = _segmean(msg_ref[...], dst_ref[...])
    h1 = _elu(agg + jnp.dot(x_ref[...], root1_ref[...],
                            preferred_element_type=jnp.float32)
              + b1_ref[...])
    h1_ref[...] = h1
    xw2_ref[...] = jnp.dot(h1, w2t_ref[...],
                           preferred_element_type=jnp.float32)


def _tc_c(msg1, dst, x, root1, b1, w2t):
    return pl.pallas_call(
        _tc_c_body,
        out_shape=(
            jax.ShapeDtypeStruct((N, K * 64), jnp.float32),
            jax.ShapeDtypeStruct((N, 32), jnp.float32),
        ),
    )(msg1, dst, x, root1, b1, w2t)


def _tc_e_body(msg_ref, dst_ref, h1_ref, root2_ref, b2_ref,
               fc1w_ref, fc1b_ref, fc2w_ref, fc2b_ref, out_ref):
    agg = _segmean(msg_ref[...], dst_ref[...])
    h2 = _elu(agg + jnp.dot(h1_ref[...], root2_ref[...],
                            preferred_element_type=jnp.float32)
              + b2_ref[...])
    y = fc1b_ref[...].reshape(1, -1)
    for n in range(N):
        y = y + jnp.dot(h2[n:n + 1, :], fc1w_ref[pl.ds(n * 64, 64), :],
                        preferred_element_type=jnp.float32)
    z = jnp.dot(y, fc2w_ref[...],
                preferred_element_type=jnp.float32) + fc2b_ref[...]
    m = jnp.max(z, axis=1, keepdims=True)
    out_ref[...] = z - (m + jnp.log(jnp.sum(jnp.exp(z - m), axis=1,
                                            keepdims=True)))


def _tc_e(msg2, dst, h1, root2, b2, fc1_w, fc1_b, fc2_w, fc2_b):
    return pl.pallas_call(
        _tc_e_body,
        out_shape=jax.ShapeDtypeStruct((1, 2), jnp.float32),
    )(msg2, dst, h1, root2, b2, fc1_w, fc1_b, fc2_w, fc2_b)


def kernel(x, edge_index, edge_attr, W1, root1, b1, W2, root2, b2,
           fc1_w, fc1_b, fc2_w, fc2_b):
    src = edge_index[0].reshape(1, E)
    dst = edge_index[1].reshape(1, E)
    eat = edge_attr.T
    w1t = W1.transpose(1, 0, 2).reshape(128, K * 32)
    w2t = W2.transpose(1, 0, 2).reshape(32, K * 64)

    xw1, basis, fidx = _tc_a(x, w1t, eat, src)
    msg1 = _sc_b(32, xw1.reshape(N * K, 32), fidx, basis)
    xw2, h1 = _tc_c(msg1, dst, x, root1, b1.reshape(1, 32), w2t)
    msg2 = _sc_b(64, xw2.reshape(N * K, 64), fidx, basis)
    return _tc_e(msg2, dst, h1, root2, b2.reshape(1, 64),
                 fc1_w, fc1_b, fc2_w, fc2_b)


# speedup vs baseline: 4.2580x; 1.0013x over previous
import functools

import jax
import jax.numpy as jnp
from jax import lax
from jax.experimental import pallas as pl
from jax.experimental.pallas import tpu as pltpu
from jax.experimental.pallas import tpu_sc as plsc

N = 12
E = 768
K = 125
S = 8
KS = 5
NW = 32
EPW = E // NW


def _tc_a_body(x_ref, w1t_ref, eat_ref, src_ref, xw_ref, basis_ref, fidx_ref):
    xw_ref[...] = jnp.dot(x_ref[...], w1t_ref[...],
                          preferred_element_type=jnp.float32)
    v = eat_ref[...] * (KS - 1.0)
    bot = jnp.clip(jnp.floor(v), 0.0, KS - 2.0)
    frac = v - bot
    boti = bot.astype(jnp.int32)
    src = src_ref[...]
    brows, irows = [], []
    for s in range(S):
        b = jnp.ones((1, E), jnp.float32)
        wi = jnp.zeros((1, E), jnp.int32)
        stride = 1
        for d in range(3):
            bit = (s >> d) & 1
            fd = frac[d:d + 1, :]
            b = b * (fd if bit else 1.0 - fd)
            wi = wi + (boti[d:d + 1, :] + bit) * stride
            stride *= KS
        brows.append(b)
        irows.append(src * K + wi)
    basis_ref[...] = jnp.concatenate(brows, axis=0).T
    fidx_ref[...] = jnp.concatenate(irows, axis=0).T


def _tc_a(x, w1t, eat, src):
    return pl.pallas_call(
        _tc_a_body,
        out_shape=(
            jax.ShapeDtypeStruct((N, K * 32), jnp.float32),
            jax.ShapeDtypeStruct((E, S), jnp.float32),
            jax.ShapeDtypeStruct((E, S), jnp.int32),
        ),
    )(x, w1t, eat, src)


def _sc_b_body(fout, xw_hbm, fidx_hbm, basis_hbm, out_hbm,
               fidx_v, basis_v, rows_v, msg_v, sem, bsem):
    wid = lax.axis_index("s") * 2 + lax.axis_index("c")
    p0 = wid * EPW * S
    bcp = pltpu.async_copy(basis_hbm.at[pl.ds(p0, EPW * S)], basis_v, bsem)
    pltpu.sync_copy(fidx_hbm.at[pl.ds(p0, EPW * S)], fidx_v)
    pltpu.async_copy(xw_hbm.at[fidx_v], rows_v, sem).wait()
    bcp.wait()
    for e in range(EPW):
        spl = [
            plsc.load_gather(basis_v, [jnp.full((16,), e * S + s, jnp.int32)])
            for s in range(S)
        ]
        for c in range(fout // 16):
            acc = jnp.zeros((16,), jnp.float32)
            for s in range(S):
                acc = acc + spl[s] * rows_v[e * S + s, pl.ds(c * 16, 16)]
            msg_v[e, pl.ds(c * 16, 16)] = acc
    pltpu.sync_copy(msg_v, out_hbm.at[pl.ds(wid * EPW, EPW), :])


def _sc_b(fout, xwflat, fidx, basis):
    mesh = plsc.VectorSubcoreMesh(core_axis_name="c", subcore_axis_name="s")
    kfn = functools.partial(
        pl.kernel,
        mesh=mesh,
        out_type=jax.ShapeDtypeStruct((E, fout), jnp.float32),
        scratch_types=[
            pltpu.VMEM((EPW * S,), jnp.int32),
            pltpu.VMEM((EPW * S,), jnp.float32),
            pltpu.VMEM((EPW * S, fout), jnp.float32),
            pltpu.VMEM((EPW, fout), jnp.float32),
            pltpu.SemaphoreType.DMA,
            pltpu.SemaphoreType.DMA,
        ],
        compiler_params=pltpu.CompilerParams(use_tc_tiling_on_sc=False,
                                             needs_layout_passes=False),
    )(functools.partial(_sc_b_body, fout))
    return kfn(xwflat, fidx.reshape(-1), basis.reshape(-1))


def _elu(a):
    return jnp.where(a > 0.0, a, jnp.exp(jnp.minimum(a, 0.0)) - 1.0)


def _segmean(msg, dst):
    onehot = jnp.where(
        lax.broadcasted_iota(jnp.int32, (N, E), 0) == dst, 1.0, 0.0)
    cnt = jnp.sum(onehot, axis=1, keepdims=True)
    agg = jnp.dot(onehot, msg, preferred_element_type=jnp.float32)
    return agg / jnp.maximum(cnt, 1.0)


def _tc_c_body(msg_ref, dst_ref, x_ref, root1_ref, b1_ref, w2t_ref,
               xw2_ref, h1_ref):
    agg = _segmean(msg_ref[...], dst_ref[...])
    h1 = _elu(agg + jnp.dot(x_ref[...], root1_ref[...],
                            preferred_element_type=jnp.float32)
              + b1_ref[...])
    h1_ref[...] = h1
    xw2_ref[...] = jnp.dot(h1, w2t_ref[...],
                           preferred_element_type=jnp.float32)


def _tc_c(msg1, dst, x, root1, b1, w2t):
    return pl.pallas_call(
        _tc_c_body,
        out_shape=(
            jax.ShapeDtypeStruct((N, K * 64), jnp.float32),
            jax.ShapeDtypeStruct((N, 32), jnp.float32),
        ),
    )(msg1, dst, x, root1, b1, w2t)


def _tc_e_body(msg_ref, dst_ref, h1_ref, root2_ref, b2_ref,
               fc1w_ref, fc1b_ref, fc2w_ref, fc2b_ref, out_ref):
    agg = _segmean(msg_ref[...], dst_ref[...])
    h2 = _elu(agg + jnp.dot(h1_ref[...], root2_ref[...],
                            preferred_element_type=jnp.float32)
              + b2_ref[...])
    y = fc1b_ref[...].reshape(1, -1)
    for n in range(N):
        y = y + jnp.dot(h2[n:n + 1, :], fc1w_ref[pl.ds(n * 64, 64), :],
                        preferred_element_type=jnp.float32)
    z = jnp.dot(y, fc2w_ref[...],
                preferred_element_type=jnp.float32) + fc2b_ref[...]
    m = jnp.max(z, axis=1, keepdims=True)
    out_ref[...] = z - (m + jnp.log(jnp.sum(jnp.exp(z - m), axis=1,
                                            keepdims=True)))


def _tc_e(msg2, dst, h1, root2, b2, fc1_w, fc1_b, fc2_w, fc2_b):
    return pl.pallas_call(
        _tc_e_body,
        out_shape=jax.ShapeDtypeStruct((1, 2), jnp.float32),
    )(msg2, dst, h1, root2, b2, fc1_w, fc1_b, fc2_w, fc2_b)


def kernel(x, edge_index, edge_attr, W1, root1, b1, W2, root2, b2,
           fc1_w, fc1_b, fc2_w, fc2_b):
    src = edge_index[0].reshape(1, E)
    dst = edge_index[1].reshape(1, E)
    eat = edge_attr.T
    w1t = W1.transpose(1, 0, 2).reshape(128, K * 32)
    w2t = W2.transpose(1, 0, 2).reshape(32, K * 64)

    xw1, basis, fidx = _tc_a(x, w1t, eat, src)
    msg1 = _sc_b(32, xw1.reshape(N * K, 32), fidx, basis)
    xw2, h1 = _tc_c(msg1, dst, x, root1, b1.reshape(1, 32), w2t)
    msg2 = _sc_b(64, xw2.reshape(N * K, 64), fidx, basis)
    return _tc_e(msg2, dst, h1, root2, b2.reshape(1, 64),
                 fc1_w, fc1_b, fc2_w, fc2_b)


# speedup vs baseline: 4.7666x; 1.1195x over previous
import functools

import jax
import jax.numpy as jnp
from jax import lax
from jax.experimental import pallas as pl
from jax.experimental.pallas import tpu as pltpu
from jax.experimental.pallas import tpu_sc as plsc

N = 12
E = 768
K = 125
S = 8
KS = 5
NW = 32
EPW = E // NW
CP = 1504
GRP = EPW * S // 16


def _sc_c_body(src_hbm, dst_hbm, attr_hbm, out_hbm, src_v, dst_v, attr_v, cl):
    wid = lax.axis_index("s") * 2 + lax.axis_index("c")
    e0 = wid * EPW
    pltpu.sync_copy(src_hbm.at[pl.ds(e0, EPW)], src_v)
    pltpu.sync_copy(dst_hbm.at[pl.ds(e0, EPW)], dst_v)
    pltpu.sync_copy(attr_hbm.at[pl.ds(e0 * 3, EPW * 3)], attr_v)

    def zero_body(i, _):
        cl[pl.ds(i * 16, 16)] = jnp.zeros((16,), jnp.float32)
        return _
    lax.fori_loop(0, N * CP // 16, zero_body, None)

    lanes = lax.broadcasted_iota(jnp.int32, (16,), 0)
    for g in range(GRP):
        p = g * 16 + lanes
        e = p >> 3
        s = p & 7
        srcp = plsc.load_gather(src_v, [e])
        dstp = plsc.load_gather(dst_v, [e])
        b = jnp.ones((16,), jnp.float32)
        wi = jnp.zeros((16,), jnp.int32)
        stride = 1
        for d in range(3):
            v = plsc.load_gather(attr_v, [e * 3 + d]) * (KS - 1.0)
            boti = jnp.minimum(v.astype(jnp.int32), KS - 2)
            frac = v - boti.astype(jnp.float32)
            bit = (s >> d) & 1
            b = b * jnp.where(bit == 1, frac, 1.0 - frac)
            wi = wi + (boti + bit) * stride
            stride *= KS
        idx = dstp * CP + srcp * K + wi
        for j in range(16):
            plsc.addupdate_scatter(cl, [idx], b, mask=lanes == j)
    pltpu.sync_copy(cl, out_hbm.at[wid])


def _sc_c(src, dst, attr_flat):
    mesh = plsc.VectorSubcoreMesh(core_axis_name="c", subcore_axis_name="s")
    kfn = functools.partial(
        pl.kernel,
        mesh=mesh,
        out_type=jax.ShapeDtypeStruct((NW, N * CP), jnp.float32),
        scratch_types=[
            pltpu.VMEM((EPW,), jnp.int32),
            pltpu.VMEM((EPW,), jnp.int32),
            pltpu.VMEM((EPW * 3,), jnp.float32),
            pltpu.VMEM((N * CP,), jnp.float32),
        ],
        compiler_params=pltpu.CompilerParams(use_tc_tiling_on_sc=False,
                                             needs_layout_passes=False),
    )(_sc_c_body)
    return kfn(src, dst, attr_flat)


def _elu(a):
    return jnp.where(a > 0.0, a, jnp.exp(jnp.minimum(a, 0.0)) - 1.0)


def _tc1_body(x_ref, w1t_ref, root1_ref, b1_ref, xw_ref, xr_ref):
    xw_ref[...] = jnp.dot(x_ref[...], w1t_ref[...],
                          preferred_element_type=jnp.float32)
    xr_ref[...] = jnp.dot(x_ref[...], root1_ref[...],
                          preferred_element_type=jnp.float32) + b1_ref[...]


def _tc1(x, w1t, root1, b1):
    return pl.pallas_call(
        _tc1_body,
        out_shape=(
            jax.ShapeDtypeStruct((N, K * 32), jnp.float32),
            jax.ShapeDtypeStruct((N, 32), jnp.float32),
        ),
    )(x, w1t, root1, b1)


def _csum_cnt(call_ref, dst_ref):
    c = call_ref[pl.ds(0, N), :]
    for w in range(1, NW):
        c = c + call_ref[pl.ds(w * N, N), :]
    onehot = jnp.where(
        lax.broadcasted_iota(jnp.int32, (N, E), 0) == dst_ref[...], 1.0, 0.0)
    cnt = jnp.maximum(jnp.sum(onehot, axis=1, keepdims=True), 1.0)
    return c, cnt


def _tc2_body(call_ref, dst_ref, xw1_ref, xr_ref, w2t_ref, root2_ref, b2_ref,
              xw2_ref, hr2_ref, c_ref, cnt_ref):
    c, cnt = _csum_cnt(call_ref, dst_ref)
    agg1 = jnp.dot(c[:, :N * K], xw1_ref[...],
                   preferred_element_type=jnp.float32) / cnt
    h1 = _elu(agg1 + xr_ref[...])
    xw2_ref[...] = jnp.dot(h1, w2t_ref[...],
                           preferred_element_type=jnp.float32)
    hr2_ref[...] = jnp.dot(h1, root2_ref[...],
                           preferred_element_type=jnp.float32) + b2_ref[...]
    c_ref[...] = c
    cnt_ref[...] = cnt


def _tc2(call, dst, xw1flat, xr, w2t, root2, b2):
    return pl.pallas_call(
        _tc2_body,
        out_shape=(
            jax.ShapeDtypeStruct((N, K * 64), jnp.float32),
            jax.ShapeDtypeStruct((N, 64), jnp.float32),
            jax.ShapeDtypeStruct((N, CP), jnp.float32),
            jax.ShapeDtypeStruct((N, 1), jnp.float32),
        ),
    )(call, dst, xw1flat, xr, w2t, root2, b2)


def _tc3_body(c_ref, cnt_ref, xw2_ref, hr2_ref,
              fc1w_ref, fc1b_ref, fc2w_ref, fc2b_ref, out_ref):
    agg2 = jnp.dot(c_ref[:, :N * K], xw2_ref[...],
                   preferred_element_type=jnp.float32) / cnt_ref[...]
    h2 = _elu(agg2 + hr2_ref[...])
    y = fc1b_ref[...]
    for n in range(N):
        y = y + jnp.dot(h2[n:n + 1, :], fc1w_ref[pl.ds(n * 64, 64), :],
                        preferred_element_type=jnp.float32)
    z = jnp.dot(y, fc2w_ref[...],
                preferred_element_type=jnp.float32) + fc2b_ref[...]
    m = jnp.max(z, axis=1, keepdims=True)
    out_ref[...] = z - (m + jnp.log(jnp.sum(jnp.exp(z - m), axis=1,
                                            keepdims=True)))


def _tc3(c, cnt, xw2flat, hr2, fc1_w, fc1_b, fc2_w, fc2_b):
    return pl.pallas_call(
        _tc3_body,
        out_shape=jax.ShapeDtypeStruct((1, 2), jnp.float32),
    )(c, cnt, xw2flat, hr2, fc1_w, fc1_b, fc2_w, fc2_b)


def kernel(x, edge_index, edge_attr, W1, root1, b1, W2, root2, b2,
           fc1_w, fc1_b, fc2_w, fc2_b):
    src = edge_index[0]
    dst = edge_index[1]
    w1t = W1.transpose(1, 0, 2).reshape(128, K * 32)
    w2t = W2.transpose(1, 0, 2).reshape(32, K * 64)

    call = _sc_c(src, dst, edge_attr.reshape(-1))
    xw1, xr = _tc1(x, w1t, root1, b1.reshape(1, 32))
    xw2, hr2, c, cnt = _tc2(call.reshape(NW * N, CP), dst.reshape(1, E),
                            xw1.reshape(N * K, 32), xr, w2t, root2,
                            b2.reshape(1, 64))
    return _tc3(c, cnt, xw2.reshape(N * K, 64), hr2,
                fc1_w, fc1_b.reshape(1, 128), fc2_w, fc2_b.reshape(1, 2))


# speedup vs baseline: 4.9171x; 1.0316x over previous
import functools

import jax
import jax.numpy as jnp
from jax import lax
from jax.experimental import pallas as pl
from jax.experimental.pallas import tpu as pltpu
from jax.experimental.pallas import tpu_sc as plsc

N = 12
E = 768
K = 125
S = 8
KS = 5
NW = 32
EPW = E // NW
KP = 128
CP = N * KP
GRP = EPW * S // 16


def _sc_c_body(src_hbm, dst_hbm, attr_hbm, zero_hbm, out_hbm,
               src_v, dst_v, attr_v, cl, zsem):
    wid = lax.axis_index("s") * 2 + lax.axis_index("c")
    e0 = wid * EPW
    zcp = pltpu.async_copy(zero_hbm, cl, zsem)
    pltpu.sync_copy(src_hbm.at[pl.ds(e0, EPW)], src_v)
    pltpu.sync_copy(dst_hbm.at[pl.ds(e0, EPW)], dst_v)
    pltpu.sync_copy(attr_hbm.at[pl.ds(e0 * 3, EPW * 3)], attr_v)
    zcp.wait()

    lanes = lax.broadcasted_iota(jnp.int32, (16,), 0)
    for g in range(GRP):
        p = g * 16 + lanes
        e = p >> 3
        s = p & 7
        srcp = plsc.load_gather(src_v, [e])
        dstp = plsc.load_gather(dst_v, [e])
        b = jnp.ones((16,), jnp.float32)
        wi = jnp.zeros((16,), jnp.int32)
        stride = 1
        for d in range(3):
            v = plsc.load_gather(attr_v, [e * 3 + d]) * (KS - 1.0)
            boti = jnp.minimum(v.astype(jnp.int32), KS - 2)
            frac = v - boti.astype(jnp.float32)
            bit = (s >> d) & 1
            b = b * jnp.where(bit == 1, frac, 1.0 - frac)
            wi = wi + (boti + bit) * stride
            stride *= KS
        idx = dstp * CP + srcp * KP + wi
        for j in range(16):
            plsc.addupdate_scatter(cl, [idx], b, mask=lanes == j)
    pltpu.sync_copy(cl, out_hbm.at[wid])


def _sc_c(src, dst, attr_flat):
    mesh = plsc.VectorSubcoreMesh(core_axis_name="c", subcore_axis_name="s")
    kfn = functools.partial(
        pl.kernel,
        mesh=mesh,
        out_type=jax.ShapeDtypeStruct((NW, N * CP), jnp.float32),
        scratch_types=[
            pltpu.VMEM((EPW,), jnp.int32),
            pltpu.VMEM((EPW,), jnp.int32),
            pltpu.VMEM((EPW * 3,), jnp.float32),
            pltpu.VMEM((N * CP,), jnp.float32),
            pltpu.SemaphoreType.DMA,
        ],
        compiler_params=pltpu.CompilerParams(use_tc_tiling_on_sc=False,
                                             needs_layout_passes=False),
    )(_sc_c_body)
    return kfn(src, dst, attr_flat, jnp.zeros((N * CP,), jnp.float32))


def _elu(a):
    return jnp.where(a > 0.0, a, jnp.exp(jnp.minimum(a, 0.0)) - 1.0)


def _tc1_body(x_ref, w1t_ref, root1_ref, b1_ref, xw_ref, xr_ref):
    xw_ref[...] = jnp.dot(x_ref[...], w1t_ref[...],
                          preferred_element_type=jnp.float32)
    xr_ref[...] = jnp.dot(x_ref[...], root1_ref[...],
                          preferred_element_type=jnp.float32) + b1_ref[...]


def _tc1(x, w1t, root1, b1):
    return pl.pallas_call(
        _tc1_body,
        out_shape=(
            jax.ShapeDtypeStruct((N, KP * 32), jnp.float32),
            jax.ShapeDtypeStruct((N, 32), jnp.float32),
        ),
    )(x, w1t, root1, b1)


def _csum_cnt(call_ref, dst_ref):
    c = call_ref[pl.ds(0, N), :]
    for w in range(1, NW):
        c = c + call_ref[pl.ds(w * N, N), :]
    onehot = jnp.where(
        lax.broadcasted_iota(jnp.int32, (N, E), 0) == dst_ref[...], 1.0, 0.0)
    cnt = jnp.maximum(jnp.sum(onehot, axis=1, keepdims=True), 1.0)
    return c, cnt


def _tcm_body(call_ref, dst_ref, xw1_ref, xr_ref, w2s_ref, root2_ref, b2_ref,
              fc1w_ref, fc1b_ref, fc2w_ref, fc2b_ref, out_ref):
    c, cnt = _csum_cnt(call_ref, dst_ref)
    agg1 = jnp.dot(c, xw1_ref[...],
                   preferred_element_type=jnp.float32) / cnt
    h1 = _elu(agg1 + xr_ref[...])
    sub = lax.broadcasted_iota(jnp.int32, (32, 32 * KP), 0)
    lane = lax.broadcasted_iota(jnp.int32, (32, 32 * KP), 1)
    onehot = jnp.where(sub == lane // KP, 1.0, 0.0)
    hrows = jnp.dot(h1, onehot, preferred_element_type=jnp.float32)
    blocks = []
    for i in range(32):
        mi = hrows[0:1, i * KP:(i + 1) * KP] * c[:, 0:KP]
        for n in range(1, N):
            mi = mi + (hrows[n:n + 1, i * KP:(i + 1) * KP]
                       * c[:, n * KP:(n + 1) * KP])
        blocks.append(mi)
    m2 = jnp.concatenate(blocks, axis=1)
    agg2 = jnp.dot(m2, w2s_ref[...],
                   preferred_element_type=jnp.float32) / cnt
    h2 = _elu(agg2 + jnp.dot(h1, root2_ref[...],
                             preferred_element_type=jnp.float32)
              + b2_ref[...])
    y = fc1b_ref[...]
    for n in range(N):
        y = y + jnp.dot(h2[n:n + 1, :], fc1w_ref[pl.ds(n * 64, 64), :],
                        preferred_element_type=jnp.float32)
    z = jnp.dot(y, fc2w_ref[...],
                preferred_element_type=jnp.float32) + fc2b_ref[...]
    m = jnp.max(z, axis=1, keepdims=True)
    out_ref[...] = z - (m + jnp.log(jnp.sum(jnp.exp(z - m), axis=1,
                                            keepdims=True)))


def _tcm(call, dst, xw1flat, xr, w2s, root2, b2, fc1_w, fc1_b, fc2_w, fc2_b):
    return pl.pallas_call(
        _tcm_body,
        out_shape=jax.ShapeDtypeStruct((1, 2), jnp.float32),
    )(call, dst, xw1flat, xr, w2s, root2, b2, fc1_w, fc1_b, fc2_w, fc2_b)


def kernel(x, edge_index, edge_attr, W1, root1, b1, W2, root2, b2,
           fc1_w, fc1_b, fc2_w, fc2_b):
    src = edge_index[0]
    dst = edge_index[1]
    w1p = jnp.pad(W1, ((0, KP - K), (0, 0), (0, 0)))
    w1t = w1p.transpose(1, 0, 2).reshape(128, KP * 32)
    w2p = jnp.pad(W2, ((0, KP - K), (0, 0), (0, 0)))
    w2s = w2p.transpose(1, 0, 2).reshape(32 * KP, 64)

    call = _sc_c(src, dst, edge_attr.reshape(-1))
    xw1, xr = _tc1(x, w1t, root1, b1.reshape(1, 32))
    return _tcm(call.reshape(NW * N, CP), dst.reshape(1, E),
                xw1.reshape(N * KP, 32), xr, w2s, root2, b2.reshape(1, 64),
                fc1_w, fc1_b.reshape(1, 128), fc2_w, fc2_b.reshape(1, 2))
